# baseline (speedup 1.0000x reference)
import numpy as np
import jax
import jax.numpy as jnp
from jax.experimental import pallas as pl
from jax.experimental.pallas import tpu as pltpu

N = 4096
D = 128
K = 17
BINS = 32
M0, M1, M2 = 64, 16, 8
RB = 1024
NB = N // RB
PAD = 8
NP2 = N + 2 * PAD
STEP = np.float32(4.0 / (BINS - 1))
INV_STEP = np.float32((BINS - 1) / 4.0)
S3 = np.float32(np.sqrt(3.0))
S15 = np.float32(np.sqrt(15.0))
HS15 = np.float32(np.sqrt(15.0) / 2.0)
HS5 = np.float32(np.sqrt(5.0) / 2.0)
INV112 = np.float32(1.0 / 1.12)
INVK = np.float32(1.0 / K)
BIG = np.float32(1e30)
MOUT = M0 + 3 * M1 + 5 * M2


def _expander(m, c):
    e = np.zeros((m, m * c), np.float32)
    for o in range(m):
        e[o, o * c:(o + 1) * c] = 1.0
    return e

def _tiler(c, m):
    e = np.zeros((c, m * c), np.float32)
    for o in range(m):
        for j in range(c):
            e[j, o * c + j] = 1.0
    return e

_R16 = _expander(M1, 3)
_S3M = _tiler(3, M1)
_R8 = _expander(M2, 5)
_S5M = _tiler(5, M2)


def _silu(x):
    return x * (1.0 / (1.0 + jnp.exp(-x)))


def _dot(a, b):
    return jnp.dot(a, b, preferred_element_type=jnp.float32)


def _dotb(a, b):
    return jnp.dot(a.astype(jnp.bfloat16), b.astype(jnp.bfloat16),
                   preferred_element_type=jnp.float32)


def _band_step(pid, fp_ref, cop_ref, w0t_ref, w1t_ref, w2lt_ref, at_ref, ct_ref,
               bt_ref, b1c_ref, w2t_ref, b2c_ref,
               r16t_ref, s3t_ref, r8t_ref, s5t_ref, out_ref):
    r0 = pid * RB
    w = RB + 2 * PAD
    ftw = fp_ref[pl.ds(r0, w), :].T
    cow = cop_ref[pl.ds(r0, w), :].T
    t0w = _dotb(w0t_ref[:, :D], ftw) + w0t_ref[:, D:D + 1]
    t1w = _dotb(w1t_ref[:, :D], ftw) + w1t_ref[:, D:D + 1]
    t2w = _dotb(w2lt_ref[:, :D], ftw) + w2lt_ref[:, D:D + 1]
    u0w = _dotb(at_ref[...], t0w)
    ucb = _dotb(ct_ref[...], ftw[:, PAD:PAD + RB])

    ccx = cow[0:1, PAD:PAD + RB]
    ccy = cow[1:2, PAD:PAD + RB]
    ccz = cow[2:3, PAD:PAD + RB]
    xs, ys, zs = [], [], []
    for t in range(K):
        xs.append(cow[0:1, t:t + RB] - ccx)
        ys.append(cow[1:2, t:t + RB] - ccy)
        zs.append(cow[2:3, t:t + RB] - ccz)
    x = jnp.concatenate(xs, 1)
    y = jnp.concatenate(ys, 1)
    z = jnp.concatenate(zs, 1)
    xx = x * x
    yy = y * y
    zz = z * z
    ns = xx + yy + zz
    norm = jnp.sqrt(jnp.where(ns == 0.0, 1.0, ns))
    valc = jax.lax.broadcasted_iota(jnp.int32, (BINS, 1), 0).astype(jnp.float32) * STEP
    dd = (norm - valc) * INV_STEP
    rad = jnp.exp(-(dd * dd)) * INV112
    sh1 = S3 * jnp.concatenate([x, y, z], 0)
    sh2 = jnp.concatenate([
        S15 * (x * y), S15 * (y * z), HS5 * (2.0 * zz - xx - yy),
        S15 * (x * z), HS15 * (xx - yy)], 0)

    t0c = jnp.concatenate([t0w[:, t:t + RB] for t in range(K)], 1)
    t1c = jnp.concatenate([t1w[:, t:t + RB] for t in range(K)], 1)
    t2c = jnp.concatenate([t2w[:, t:t + RB] for t in range(K)], 1)
    u0c = jnp.concatenate([u0w[:, t:t + RB] for t in range(K)], 1)
    ucc = jnp.concatenate([ucb] * K, 1)

    pre = u0c + ucc + b1c_ref[...] + _dotb(bt_ref[...], rad)
    h = _silu(pre)
    mix = _dotb(w2t_ref[...], h) + b2c_ref[...]
    e0 = t0c * mix[:M0, :]
    t1m = t1c * mix[M0:M0 + M1, :]
    e1 = _dotb(r16t_ref[...], t1m) * _dotb(s3t_ref[...], sh1)
    t2m = t2c * mix[M0 + M1:, :]
    e2 = _dotb(r8t_ref[...], t2m) * _dotb(s5t_ref[...], sh2)
    o0 = e0[:, 0:RB]
    o1 = e1[:, 0:RB]
    o2 = e2[:, 0:RB]
    for t in range(1, K):
        o0 = o0 + e0[:, t * RB:(t + 1) * RB]
        o1 = o1 + e1[:, t * RB:(t + 1) * RB]
        o2 = o2 + e2[:, t * RB:(t + 1) * RB]
    out_ref[pl.ds(r0, RB), :] = (jnp.concatenate([o0, o1, o2], 0) * INVK).T


def _geom_rows(vec):
    x = vec[:, 0:1]
    y = vec[:, 1:2]
    z = vec[:, 2:3]
    ns = x * x + y * y + z * z
    norm = jnp.sqrt(jnp.where(ns == 0.0, 1.0, ns))
    vals = jax.lax.broadcasted_iota(jnp.int32, (1, BINS), 1).astype(jnp.float32) * STEP
    dd = (norm - vals) * INV_STEP
    rad = jnp.exp(-(dd * dd)) * INV112
    sh1 = S3 * vec
    sh2 = jnp.concatenate([
        S15 * (x * y), S15 * (y * z), HS5 * (2.0 * z * z - x * x - y * y),
        S15 * (x * z), HS15 * (x * x - y * y)], axis=1)
    return rad, sh1, sh2


def _boundary_step(co_ref, f_ref, w0_ref, w1_ref, w2l_ref, a_ref, c_ref,
                   b_ref, b1_ref, w2_ref, b2_ref, r16_ref, s3_ref, r8_ref, s5_ref,
                   out_ref):
    cot = co_ref[...].T
    bco = jnp.concatenate([co_ref[0:PAD, :], co_ref[N - PAD:N, :]], axis=0)
    d2 = jnp.zeros((2 * PAD, N), jnp.float32)
    for c in range(3):
        diff = cot[c:c + 1, :] - bco[:, c:c + 1]
        d2 = d2 + diff * diff
    j2 = jax.lax.broadcasted_iota(jnp.int32, (2 * PAD, N), 1)
    r1 = jax.lax.broadcasted_iota(jnp.int32, (2 * PAD, 1), 0)
    low = jnp.where(r1 < PAD, r1 + PAD, -1)
    high = jnp.where(r1 < PAD, N + 1, (N - 24) + r1)
    d2m = jnp.where((j2 <= low) | (j2 >= high), BIG, d2)
    spats = []
    for _ in range(PAD):
        m = jnp.min(d2m, axis=1, keepdims=True)
        am = jnp.min(jnp.where(d2m == m, j2, N), axis=1, keepdims=True)
        spats.append(am)
        d2m = jnp.where(j2 == am, BIG, d2m)
    spat = jnp.concatenate(spats, axis=1)
    tt = jax.lax.broadcasted_iota(jnp.int32, (2 * PAD, K), 1)
    rr = jax.lax.broadcasted_iota(jnp.int32, (2 * PAD, K), 0)
    ig = jnp.where(rr < PAD, rr, (N - 2 * PAD) + rr)
    fixedcnt = jnp.where(rr < PAD, rr + 9, 24 - rr)
    base = jnp.where(rr < PAD, tt, ig - PAD + tt)
    s_idx = tt - fixedcnt
    gath = jnp.zeros((2 * PAD, K), jnp.int32)
    for s in range(PAD):
        gath = gath + jnp.where(s_idx == s, spat[:, s:s + 1], 0)
    nei = jnp.where(tt < fixedcnt, base, gath)
    ohs = []
    for t in range(K):
        ohs.append((j2 == nei[:, t:t + 1]).astype(jnp.float32))
    oh = jnp.concatenate(ohs, axis=0)
    gf = _dotb(oh, f_ref[...])
    gco = _dot(oh, co_ref[...])
    g0 = _dotb(gf, w0_ref[:D, :]) + w0_ref[D:D + 1, :]
    g1 = _dotb(gf, w1_ref[:D, :]) + w1_ref[D:D + 1, :]
    g2 = _dotb(gf, w2l_ref[:D, :]) + w2l_ref[D:D + 1, :]
    gu = _dotb(g0, a_ref[...])
    fc = jnp.concatenate([f_ref[0:PAD, :], f_ref[N - PAD:N, :]], axis=0)
    ucb = _dotb(fc, c_ref[...])
    vec = gco - jnp.concatenate([bco] * K, 0)
    rad, sh1, sh2 = _geom_rows(vec)
    pre = (gu + jnp.concatenate([ucb] * K, 0) + b1_ref[...] + _dotb(rad, b_ref[...]))
    h = _silu(pre)
    mix = _dotb(h, w2_ref[...]) + b2_ref[...]
    e0 = g0 * mix[:, :M0]
    t1m = g1 * mix[:, M0:M0 + M1]
    e1 = _dotb(t1m, r16_ref[...]) * _dotb(sh1, s3_ref[...])
    t2m = g2 * mix[:, M0 + M1:]
    e2 = _dotb(t2m, r8_ref[...]) * _dotb(sh2, s5_ref[...])
    o0 = jnp.sum(e0.reshape(K, 2 * PAD, M0), axis=0) * INVK
    o1 = jnp.sum(e1.reshape(K, 2 * PAD, 3 * M1), axis=0) * INVK
    o2 = jnp.sum(e2.reshape(K, 2 * PAD, 5 * M2), axis=0) * INVK
    res = jnp.concatenate([o0, o1, o2], axis=1)
    out_ref[0:PAD, :] = res[0:PAD, :]
    out_ref[N - PAD:N, :] = res[PAD:2 * PAD, :]


def _fused_kernel(fp_ref, cop_ref, f_ref, co_ref,
                  w0t_ref, w1t_ref, w2lt_ref, at_ref, ct_ref,
                  bt_ref, b1c_ref, w2t_ref, b2c_ref,
                  r16t_ref, s3t_ref, r8t_ref, s5t_ref,
                  w0_ref, w1_ref, w2l_ref, a_ref, c_ref,
                  b_ref, b1_ref, w2_ref, b2_ref,
                  r16_ref, s3_ref, r8_ref, s5_ref,
                  out_ref):
    pid = pl.program_id(0)

    @pl.when(pid < NB)
    def _band():
        _band_step(pid, fp_ref, cop_ref, w0t_ref, w1t_ref, w2lt_ref, at_ref,
                   ct_ref, bt_ref, b1c_ref, w2t_ref, b2c_ref,
                   r16t_ref, s3t_ref, r8t_ref, s5t_ref, out_ref)

    @pl.when(pid == NB)
    def _bnd():
        _boundary_step(co_ref, f_ref, w0_ref, w1_ref, w2l_ref, a_ref, c_ref,
                       b_ref, b1_ref, w2_ref, b2_ref,
                       r16_ref, s3_ref, r8_ref, s5_ref, out_ref)


def kernel(features, coord, mask, lin_w0, lin_w1, lin_w2, mlp_w1, mlp_b1, mlp_w2, mlp_b2):
    f32 = jnp.float32
    features = features.astype(f32)
    coord = coord.astype(f32)
    a_w = mlp_w1[:M0, :]
    b_w = mlp_w1[M0:M0 + BINS, :]
    c_w = mlp_w1[M0 + BINS:, :]
    b1 = mlp_b1.reshape(1, BINS)
    b2 = mlp_b2.reshape(1, M0 + M1 + M2)
    rpad = ((PAD, PAD), (0, 0))
    fpad = jnp.pad(features, rpad)
    cop = jnp.pad(coord, rpad)

    wcol = lambda shp: pl.BlockSpec(shp, lambda i: tuple(0 for _ in shp))
    out = pl.pallas_call(
        _fused_kernel,
        grid=(NB + 1,),
        in_specs=[
            wcol((NP2, D)), wcol((NP2, 3)), wcol((N, D)), wcol((N, 3)),
            wcol((M0, D + 1)), wcol((M1, D + 1)), wcol((M2, D + 1)),
            wcol((BINS, M0)), wcol((BINS, D)),
            wcol((BINS, BINS)), wcol((BINS, 1)),
            wcol((M0 + M1 + M2, BINS)), wcol((M0 + M1 + M2, 1)),
            wcol((3 * M1, M1)), wcol((3 * M1, 3)),
            wcol((5 * M2, M2)), wcol((5 * M2, 5)),
            wcol((D + 1, M0)), wcol((D + 1, M1)), wcol((D + 1, M2)),
            wcol((M0, BINS)), wcol((D, BINS)),
            wcol((BINS, BINS)), wcol((1, BINS)),
            wcol((BINS, M0 + M1 + M2)), wcol((1, M0 + M1 + M2)),
            wcol((M1, 3 * M1)), wcol((3, 3 * M1)),
            wcol((M2, 5 * M2)), wcol((5, 5 * M2)),
        ],
        out_specs=wcol((N, MOUT)),
        out_shape=jax.ShapeDtypeStruct((N, MOUT), f32),
    )(fpad, cop, features, coord,
      lin_w0.T, lin_w1.T, lin_w2.T, a_w.T, c_w.T,
      b_w.T, mlp_b1.reshape(BINS, 1), mlp_w2.T, mlp_b2.reshape(M0 + M1 + M2, 1),
      _R16.T, _S3M.T, _R8.T, _S5M.T,
      lin_w0, lin_w1, lin_w2, a_w, c_w,
      b_w, b1, mlp_w2, b2,
      _R16, _S3M, _R8, _S5M)
    return out


# speedup vs baseline: 1.0012x; 1.0012x over previous
import numpy as np
import jax
import jax.numpy as jnp
from jax.experimental import pallas as pl
from jax.experimental.pallas import tpu as pltpu

N = 4096
D = 128
K = 17
BINS = 32
M0, M1, M2 = 64, 16, 8
RB = 1024
NB = N // RB
PAD = 8
NP2 = N + 2 * PAD
STEP = np.float32(4.0 / (BINS - 1))
INV_STEP = np.float32((BINS - 1) / 4.0)
S3 = np.float32(np.sqrt(3.0))
S15 = np.float32(np.sqrt(15.0))
HS15 = np.float32(np.sqrt(15.0) / 2.0)
HS5 = np.float32(np.sqrt(5.0) / 2.0)
INV112 = np.float32(1.0 / 1.12)
INVK = np.float32(1.0 / K)
BIG = np.float32(1e30)
MOUT = M0 + 3 * M1 + 5 * M2


def _expander(m, c):
    e = np.zeros((m, m * c), np.float32)
    for o in range(m):
        e[o, o * c:(o + 1) * c] = 1.0
    return e

def _tiler(c, m):
    e = np.zeros((c, m * c), np.float32)
    for o in range(m):
        for j in range(c):
            e[j, o * c + j] = 1.0
    return e

_R16 = _expander(M1, 3)
_S3M = _tiler(3, M1)
_R8 = _expander(M2, 5)
_S5M = _tiler(5, M2)


def _silu(x):
    return x * (1.0 / (1.0 + jnp.exp(-x)))


def _dot(a, b):
    return jnp.dot(a, b, preferred_element_type=jnp.float32)


def _dot(a, b):
    return jnp.dot(a.astype(jnp.bfloat16), b.astype(jnp.bfloat16),
                   preferred_element_type=jnp.float32)


def _band_step(pid, fp_ref, cop_ref, w0t_ref, w1t_ref, w2lt_ref, at_ref, ct_ref,
               bt_ref, b1c_ref, w2t_ref, b2c_ref,
               r16t_ref, s3t_ref, r8t_ref, s5t_ref, out_ref):
    r0 = pid * RB
    w = RB + 2 * PAD
    ftw = fp_ref[pl.ds(r0, w), :].T
    cow = cop_ref[pl.ds(r0, w), :].T
    t0w = _dot(w0t_ref[:, :D], ftw) + w0t_ref[:, D:D + 1]
    t1w = _dot(w1t_ref[:, :D], ftw) + w1t_ref[:, D:D + 1]
    t2w = _dot(w2lt_ref[:, :D], ftw) + w2lt_ref[:, D:D + 1]
    u0w = _dot(at_ref[...], t0w)
    ucb = _dot(ct_ref[...], ftw[:, PAD:PAD + RB])

    ccx = cow[0:1, PAD:PAD + RB]
    ccy = cow[1:2, PAD:PAD + RB]
    ccz = cow[2:3, PAD:PAD + RB]
    xs, ys, zs = [], [], []
    for t in range(K):
        xs.append(cow[0:1, t:t + RB] - ccx)
        ys.append(cow[1:2, t:t + RB] - ccy)
        zs.append(cow[2:3, t:t + RB] - ccz)
    x = jnp.concatenate(xs, 1)
    y = jnp.concatenate(ys, 1)
    z = jnp.concatenate(zs, 1)
    xx = x * x
    yy = y * y
    zz = z * z
    ns = xx + yy + zz
    norm = jnp.sqrt(jnp.where(ns == 0.0, 1.0, ns))
    valc = jax.lax.broadcasted_iota(jnp.int32, (BINS, 1), 0).astype(jnp.float32) * STEP
    dd = (norm - valc) * INV_STEP
    rad = jnp.exp(-(dd * dd)) * INV112
    sh1 = S3 * jnp.concatenate([x, y, z], 0)
    sh2 = jnp.concatenate([
        S15 * (x * y), S15 * (y * z), HS5 * (2.0 * zz - xx - yy),
        S15 * (x * z), HS15 * (xx - yy)], 0)

    t0c = jnp.concatenate([t0w[:, t:t + RB] for t in range(K)], 1)
    t1c = jnp.concatenate([t1w[:, t:t + RB] for t in range(K)], 1)
    t2c = jnp.concatenate([t2w[:, t:t + RB] for t in range(K)], 1)
    u0c = jnp.concatenate([u0w[:, t:t + RB] for t in range(K)], 1)
    ucc = jnp.concatenate([ucb] * K, 1)

    pre = u0c + ucc + b1c_ref[...] + _dot(bt_ref[...], rad)
    h = _silu(pre)
    mix = _dot(w2t_ref[...], h) + b2c_ref[...]
    e0 = t0c * mix[:M0, :]
    t1m = t1c * mix[M0:M0 + M1, :]
    e1 = _dot(r16t_ref[...], t1m) * _dot(s3t_ref[...], sh1)
    t2m = t2c * mix[M0 + M1:, :]
    e2 = _dot(r8t_ref[...], t2m) * _dot(s5t_ref[...], sh2)
    o0 = e0[:, 0:RB]
    o1 = e1[:, 0:RB]
    o2 = e2[:, 0:RB]
    for t in range(1, K):
        o0 = o0 + e0[:, t * RB:(t + 1) * RB]
        o1 = o1 + e1[:, t * RB:(t + 1) * RB]
        o2 = o2 + e2[:, t * RB:(t + 1) * RB]
    out_ref[pl.ds(r0, RB), :] = (jnp.concatenate([o0, o1, o2], 0) * INVK).T


def _geom_rows(vec):
    x = vec[:, 0:1]
    y = vec[:, 1:2]
    z = vec[:, 2:3]
    ns = x * x + y * y + z * z
    norm = jnp.sqrt(jnp.where(ns == 0.0, 1.0, ns))
    vals = jax.lax.broadcasted_iota(jnp.int32, (1, BINS), 1).astype(jnp.float32) * STEP
    dd = (norm - vals) * INV_STEP
    rad = jnp.exp(-(dd * dd)) * INV112
    sh1 = S3 * vec
    sh2 = jnp.concatenate([
        S15 * (x * y), S15 * (y * z), HS5 * (2.0 * z * z - x * x - y * y),
        S15 * (x * z), HS15 * (x * x - y * y)], axis=1)
    return rad, sh1, sh2


def _boundary_step(co_ref, f_ref, w0_ref, w1_ref, w2l_ref, a_ref, c_ref,
                   b_ref, b1_ref, w2_ref, b2_ref, r16_ref, s3_ref, r8_ref, s5_ref,
                   out_ref):
    cot = co_ref[...].T
    bco = jnp.concatenate([co_ref[0:PAD, :], co_ref[N - PAD:N, :]], axis=0)
    d2 = jnp.zeros((2 * PAD, N), jnp.float32)
    for c in range(3):
        diff = cot[c:c + 1, :] - bco[:, c:c + 1]
        d2 = d2 + diff * diff
    j2 = jax.lax.broadcasted_iota(jnp.int32, (2 * PAD, N), 1)
    r1 = jax.lax.broadcasted_iota(jnp.int32, (2 * PAD, 1), 0)
    low = jnp.where(r1 < PAD, r1 + PAD, -1)
    high = jnp.where(r1 < PAD, N + 1, (N - 24) + r1)
    d2m = jnp.where((j2 <= low) | (j2 >= high), BIG, d2)
    spats = []
    for _ in range(PAD):
        m = jnp.min(d2m, axis=1, keepdims=True)
        am = jnp.min(jnp.where(d2m == m, j2, N), axis=1, keepdims=True)
        spats.append(am)
        d2m = jnp.where(j2 == am, BIG, d2m)
    spat = jnp.concatenate(spats, axis=1)
    tt = jax.lax.broadcasted_iota(jnp.int32, (2 * PAD, K), 1)
    rr = jax.lax.broadcasted_iota(jnp.int32, (2 * PAD, K), 0)
    ig = jnp.where(rr < PAD, rr, (N - 2 * PAD) + rr)
    fixedcnt = jnp.where(rr < PAD, rr + 9, 24 - rr)
    base = jnp.where(rr < PAD, tt, ig - PAD + tt)
    s_idx = tt - fixedcnt
    gath = jnp.zeros((2 * PAD, K), jnp.int32)
    for s in range(PAD):
        gath = gath + jnp.where(s_idx == s, spat[:, s:s + 1], 0)
    nei = jnp.where(tt < fixedcnt, base, gath)
    ohs = []
    for t in range(K):
        ohs.append((j2 == nei[:, t:t + 1]).astype(jnp.float32))
    oh = jnp.concatenate(ohs, axis=0)
    gf = _dot(oh, f_ref[...])
    gco = _dot(oh, co_ref[...])
    g0 = _dot(gf, w0_ref[:D, :]) + w0_ref[D:D + 1, :]
    g1 = _dot(gf, w1_ref[:D, :]) + w1_ref[D:D + 1, :]
    g2 = _dot(gf, w2l_ref[:D, :]) + w2l_ref[D:D + 1, :]
    gu = _dot(g0, a_ref[...])
    fc = jnp.concatenate([f_ref[0:PAD, :], f_ref[N - PAD:N, :]], axis=0)
    ucb = _dot(fc, c_ref[...])
    vec = gco - jnp.concatenate([bco] * K, 0)
    rad, sh1, sh2 = _geom_rows(vec)
    pre = (gu + jnp.concatenate([ucb] * K, 0) + b1_ref[...] + _dot(rad, b_ref[...]))
    h = _silu(pre)
    mix = _dot(h, w2_ref[...]) + b2_ref[...]
    e0 = g0 * mix[:, :M0]
    t1m = g1 * mix[:, M0:M0 + M1]
    e1 = _dot(t1m, r16_ref[...]) * _dot(sh1, s3_ref[...])
    t2m = g2 * mix[:, M0 + M1:]
    e2 = _dot(t2m, r8_ref[...]) * _dot(sh2, s5_ref[...])
    o0 = jnp.sum(e0.reshape(K, 2 * PAD, M0), axis=0) * INVK
    o1 = jnp.sum(e1.reshape(K, 2 * PAD, 3 * M1), axis=0) * INVK
    o2 = jnp.sum(e2.reshape(K, 2 * PAD, 5 * M2), axis=0) * INVK
    res = jnp.concatenate([o0, o1, o2], axis=1)
    out_ref[0:PAD, :] = res[0:PAD, :]
    out_ref[N - PAD:N, :] = res[PAD:2 * PAD, :]


def _fused_kernel(fp_ref, cop_ref, f_ref, co_ref,
                  w0t_ref, w1t_ref, w2lt_ref, at_ref, ct_ref,
                  bt_ref, b1c_ref, w2t_ref, b2c_ref,
                  r16t_ref, s3t_ref, r8t_ref, s5t_ref,
                  w0_ref, w1_ref, w2l_ref, a_ref, c_ref,
                  b_ref, b1_ref, w2_ref, b2_ref,
                  r16_ref, s3_ref, r8_ref, s5_ref,
                  out_ref):
    pid = pl.program_id(0)

    @pl.when(pid < NB)
    def _band():
        _band_step(pid, fp_ref, cop_ref, w0t_ref, w1t_ref, w2lt_ref, at_ref,
                   ct_ref, bt_ref, b1c_ref, w2t_ref, b2c_ref,
                   r16t_ref, s3t_ref, r8t_ref, s5t_ref, out_ref)

    @pl.when(pid == NB)
    def _bnd():
        _boundary_step(co_ref, f_ref, w0_ref, w1_ref, w2l_ref, a_ref, c_ref,
                       b_ref, b1_ref, w2_ref, b2_ref,
                       r16_ref, s3_ref, r8_ref, s5_ref, out_ref)


def kernel(features, coord, mask, lin_w0, lin_w1, lin_w2, mlp_w1, mlp_b1, mlp_w2, mlp_b2):
    f32 = jnp.float32
    features = features.astype(f32)
    coord = coord.astype(f32)
    a_w = mlp_w1[:M0, :]
    b_w = mlp_w1[M0:M0 + BINS, :]
    c_w = mlp_w1[M0 + BINS:, :]
    b1 = mlp_b1.reshape(1, BINS)
    b2 = mlp_b2.reshape(1, M0 + M1 + M2)
    rpad = ((PAD, PAD), (0, 0))
    fpad = jnp.pad(features, rpad)
    cop = jnp.pad(coord, rpad)

    wcol = lambda shp: pl.BlockSpec(shp, lambda i: tuple(0 for _ in shp))
    out = pl.pallas_call(
        _fused_kernel,
        grid=(NB + 1,),
        in_specs=[
            wcol((NP2, D)), wcol((NP2, 3)), wcol((N, D)), wcol((N, 3)),
            wcol((M0, D + 1)), wcol((M1, D + 1)), wcol((M2, D + 1)),
            wcol((BINS, M0)), wcol((BINS, D)),
            wcol((BINS, BINS)), wcol((BINS, 1)),
            wcol((M0 + M1 + M2, BINS)), wcol((M0 + M1 + M2, 1)),
            wcol((3 * M1, M1)), wcol((3 * M1, 3)),
            wcol((5 * M2, M2)), wcol((5 * M2, 5)),
            wcol((D + 1, M0)), wcol((D + 1, M1)), wcol((D + 1, M2)),
            wcol((M0, BINS)), wcol((D, BINS)),
            wcol((BINS, BINS)), wcol((1, BINS)),
            wcol((BINS, M0 + M1 + M2)), wcol((1, M0 + M1 + M2)),
            wcol((M1, 3 * M1)), wcol((3, 3 * M1)),
            wcol((M2, 5 * M2)), wcol((5, 5 * M2)),
        ],
        out_specs=wcol((N, MOUT)),
        out_shape=jax.ShapeDtypeStruct((N, MOUT), f32),
    )(fpad, cop, features, coord,
      lin_w0.T, lin_w1.T, lin_w2.T, a_w.T, c_w.T,
      b_w.T, mlp_b1.reshape(BINS, 1), mlp_w2.T, mlp_b2.reshape(M0 + M1 + M2, 1),
      _R16.T, _S3M.T, _R8.T, _S5M.T,
      lin_w0, lin_w1, lin_w2, a_w, c_w,
      b_w, b1, mlp_w2, b2,
      _R16, _S3M, _R8, _S5M)
    return out


# speedup vs baseline: 1.0314x; 1.0302x over previous
import numpy as np
import jax
import jax.numpy as jnp
from jax.experimental import pallas as pl

N = 4096
D = 128
K = 17
BINS = 32
M0, M1, M2 = 64, 16, 8
RB = 1024
NB = N // RB
PAD = 8
NP2 = N + 2 * PAD
STEP = np.float32(4.0 / (BINS - 1))
INV_STEP = np.float32((BINS - 1) / 4.0)
S3 = np.float32(np.sqrt(3.0))
S15 = np.float32(np.sqrt(15.0))
HS15 = np.float32(np.sqrt(15.0) / 2.0)
HS5 = np.float32(np.sqrt(5.0) / 2.0)
INV112 = np.float32(1.0 / 1.12)
INVK = np.float32(1.0 / K)
BIG = np.float32(1e30)
MOUT = M0 + 3 * M1 + 5 * M2


def _expander(m, c):
    e = np.zeros((m, m * c), np.float32)
    for o in range(m):
        e[o, o * c:(o + 1) * c] = 1.0
    return e

def _tiler(c, m):
    e = np.zeros((c, m * c), np.float32)
    for o in range(m):
        for j in range(c):
            e[j, o * c + j] = 1.0
    return e

_R16 = _expander(M1, 3)
_S3M = _tiler(3, M1)
_R8 = _expander(M2, 5)
_S5M = _tiler(5, M2)


def _silu(x):
    return x * (1.0 / (1.0 + jnp.exp(-x)))


def _dot(a, b):
    return jnp.dot(a, b, preferred_element_type=jnp.float32)


def _band_step(pid, fp_ref, cop_ref, w0t_ref, w1t_ref, w2lt_ref, at_ref, ct_ref,
               bt_ref, b1c_ref, w2t_ref, b2c_ref,
               r16t_ref, s3t_ref, r8t_ref, s5t_ref, out_ref):
    r0 = pid * RB
    w = RB + 2 * PAD
    ftw = fp_ref[pl.ds(r0, w), :].T
    cow = cop_ref[pl.ds(r0, w), :].T
    t0w = _dot(w0t_ref[:, :D], ftw) + w0t_ref[:, D:D + 1]
    t1w = _dot(w1t_ref[:, :D], ftw) + w1t_ref[:, D:D + 1]
    t2w = _dot(w2lt_ref[:, :D], ftw) + w2lt_ref[:, D:D + 1]
    u0w = _dot(at_ref[...], t0w)
    ucb = _dot(ct_ref[...], ftw[:, PAD:PAD + RB])

    ccx = cow[0:1, PAD:PAD + RB]
    ccy = cow[1:2, PAD:PAD + RB]
    ccz = cow[2:3, PAD:PAD + RB]
    xs, ys, zs = [], [], []
    for t in range(K):
        xs.append(cow[0:1, t:t + RB] - ccx)
        ys.append(cow[1:2, t:t + RB] - ccy)
        zs.append(cow[2:3, t:t + RB] - ccz)
    x = jnp.concatenate(xs, 1)
    y = jnp.concatenate(ys, 1)
    z = jnp.concatenate(zs, 1)
    xx = x * x
    yy = y * y
    zz = z * z
    ns = xx + yy + zz
    norm = jnp.sqrt(jnp.where(ns == 0.0, 1.0, ns))
    valc = jax.lax.broadcasted_iota(jnp.int32, (BINS, 1), 0).astype(jnp.float32) * STEP
    dd = (norm - valc) * INV_STEP
    rad = jnp.exp(-(dd * dd)) * INV112
    sh1 = S3 * jnp.concatenate([x, y, z], 0)
    sh2 = jnp.concatenate([
        S15 * (x * y), S15 * (y * z), HS5 * (2.0 * zz - xx - yy),
        S15 * (x * z), HS15 * (xx - yy)], 0)

    t0c = jnp.concatenate([t0w[:, t:t + RB] for t in range(K)], 1)
    t1c = jnp.concatenate([t1w[:, t:t + RB] for t in range(K)], 1)
    t2c = jnp.concatenate([t2w[:, t:t + RB] for t in range(K)], 1)
    u0c = jnp.concatenate([u0w[:, t:t + RB] for t in range(K)], 1)
    ucc = jnp.concatenate([ucb] * K, 1)

    pre = u0c + ucc + b1c_ref[...] + _dot(bt_ref[...], rad)
    h = _silu(pre)
    mix = _dot(w2t_ref[...], h) + b2c_ref[...]
    e0 = t0c * mix[:M0, :]
    t1m = t1c * mix[M0:M0 + M1, :]
    e1 = _dot(r16t_ref[...], t1m) * _dot(s3t_ref[...], sh1)
    t2m = t2c * mix[M0 + M1:, :]
    e2 = _dot(r8t_ref[...], t2m) * _dot(s5t_ref[...], sh2)
    o0 = e0[:, 0:RB]
    o1 = e1[:, 0:RB]
    o2 = e2[:, 0:RB]
    for t in range(1, K):
        o0 = o0 + e0[:, t * RB:(t + 1) * RB]
        o1 = o1 + e1[:, t * RB:(t + 1) * RB]
        o2 = o2 + e2[:, t * RB:(t + 1) * RB]
    out_ref[pl.ds(r0, RB), :] = (jnp.concatenate([o0, o1, o2], 0) * INVK).T


def _geom_rows(vec):
    x = vec[:, 0:1]
    y = vec[:, 1:2]
    z = vec[:, 2:3]
    ns = x * x + y * y + z * z
    norm = jnp.sqrt(jnp.where(ns == 0.0, 1.0, ns))
    vals = jax.lax.broadcasted_iota(jnp.int32, (1, BINS), 1).astype(jnp.float32) * STEP
    dd = (norm - vals) * INV_STEP
    rad = jnp.exp(-(dd * dd)) * INV112
    sh1 = S3 * vec
    sh2 = jnp.concatenate([
        S15 * (x * y), S15 * (y * z), HS5 * (2.0 * z * z - x * x - y * y),
        S15 * (x * z), HS15 * (x * x - y * y)], axis=1)
    return rad, sh1, sh2


def _boundary_step(co_ref, f_ref, w0_ref, w1_ref, w2l_ref, a_ref, c_ref,
                   b_ref, b1_ref, w2_ref, b2_ref, r16_ref, s3_ref, r8_ref, s5_ref,
                   out_ref):
    cot = co_ref[...].T
    bco = jnp.concatenate([co_ref[0:PAD, :], co_ref[N - PAD:N, :]], axis=0)
    d2 = jnp.zeros((2 * PAD, N), jnp.float32)
    for c in range(3):
        diff = cot[c:c + 1, :] - bco[:, c:c + 1]
        d2 = d2 + diff * diff
    j2 = jax.lax.broadcasted_iota(jnp.int32, (2 * PAD, N), 1)
    r1 = jax.lax.broadcasted_iota(jnp.int32, (2 * PAD, 1), 0)
    low = jnp.where(r1 < PAD, r1 + PAD, -1)
    high = jnp.where(r1 < PAD, N + 1, (N - 24) + r1)
    d2m = jnp.where((j2 <= low) | (j2 >= high), BIG, d2)
    spats = []
    for _ in range(PAD):
        m = jnp.min(d2m, axis=1, keepdims=True)
        am = jnp.min(jnp.where(d2m == m, j2, N), axis=1, keepdims=True)
        spats.append(am)
        d2m = jnp.where(j2 == am, BIG, d2m)
    spat = jnp.concatenate(spats, axis=1)
    tt = jax.lax.broadcasted_iota(jnp.int32, (2 * PAD, K), 1)
    rr = jax.lax.broadcasted_iota(jnp.int32, (2 * PAD, K), 0)
    ig = jnp.where(rr < PAD, rr, (N - 2 * PAD) + rr)
    fixedcnt = jnp.where(rr < PAD, rr + 9, 24 - rr)
    base = jnp.where(rr < PAD, tt, ig - PAD + tt)
    s_idx = tt - fixedcnt
    gath = jnp.zeros((2 * PAD, K), jnp.int32)
    for s in range(PAD):
        gath = gath + jnp.where(s_idx == s, spat[:, s:s + 1], 0)
    nei = jnp.where(tt < fixedcnt, base, gath)
    ohs = []
    for t in range(K):
        ohs.append((j2 == nei[:, t:t + 1]).astype(jnp.float32))
    oh = jnp.concatenate(ohs, axis=0)
    gf = _dot(oh, f_ref[...])
    gco = _dot(oh, co_ref[...])
    g0 = _dot(gf, w0_ref[:D, :]) + w0_ref[D:D + 1, :]
    g1 = _dot(gf, w1_ref[:D, :]) + w1_ref[D:D + 1, :]
    g2 = _dot(gf, w2l_ref[:D, :]) + w2l_ref[D:D + 1, :]
    gu = _dot(g0, a_ref[...])
    fc = jnp.concatenate([f_ref[0:PAD, :], f_ref[N - PAD:N, :]], axis=0)
    ucb = _dot(fc, c_ref[...])
    vec = gco - jnp.concatenate([bco] * K, 0)
    rad, sh1, sh2 = _geom_rows(vec)
    pre = (gu + jnp.concatenate([ucb] * K, 0) + b1_ref[...] + _dot(rad, b_ref[...]))
    h = _silu(pre)
    mix = _dot(h, w2_ref[...]) + b2_ref[...]
    e0 = g0 * mix[:, :M0]
    t1m = g1 * mix[:, M0:M0 + M1]
    e1 = _dot(t1m, r16_ref[...]) * _dot(sh1, s3_ref[...])
    t2m = g2 * mix[:, M0 + M1:]
    e2 = _dot(t2m, r8_ref[...]) * _dot(sh2, s5_ref[...])
    o0 = jnp.sum(e0.reshape(K, 2 * PAD, M0), axis=0) * INVK
    o1 = jnp.sum(e1.reshape(K, 2 * PAD, 3 * M1), axis=0) * INVK
    o2 = jnp.sum(e2.reshape(K, 2 * PAD, 5 * M2), axis=0) * INVK
    res = jnp.concatenate([o0, o1, o2], axis=1)
    out_ref[0:PAD, :] = res[0:PAD, :]
    out_ref[N - PAD:N, :] = res[PAD:2 * PAD, :]


def _fused_kernel(fp_ref, cop_ref, f_ref, co_ref,
                  w0t_ref, w1t_ref, w2lt_ref, at_ref, ct_ref,
                  bt_ref, b1c_ref, w2t_ref, b2c_ref,
                  r16t_ref, s3t_ref, r8t_ref, s5t_ref,
                  w0_ref, w1_ref, w2l_ref, a_ref, c_ref,
                  b_ref, b1_ref, w2_ref, b2_ref,
                  r16_ref, s3_ref, r8_ref, s5_ref,
                  out_ref):
    pid = pl.program_id(0)

    @pl.when(pid < NB)
    def _band():
        _band_step(pid, fp_ref, cop_ref, w0t_ref, w1t_ref, w2lt_ref, at_ref,
                   ct_ref, bt_ref, b1c_ref, w2t_ref, b2c_ref,
                   r16t_ref, s3t_ref, r8t_ref, s5t_ref, out_ref)

    @pl.when(pid == NB)
    def _bnd():
        _boundary_step(co_ref, f_ref, w0_ref, w1_ref, w2l_ref, a_ref, c_ref,
                       b_ref, b1_ref, w2_ref, b2_ref,
                       r16_ref, s3_ref, r8_ref, s5_ref, out_ref)


def kernel(features, coord, mask, lin_w0, lin_w1, lin_w2, mlp_w1, mlp_b1, mlp_w2, mlp_b2):
    f32 = jnp.float32
    features = features.astype(f32)
    coord = coord.astype(f32)
    a_w = mlp_w1[:M0, :]
    b_w = mlp_w1[M0:M0 + BINS, :]
    c_w = mlp_w1[M0 + BINS:, :]
    b1 = mlp_b1.reshape(1, BINS)
    b2 = mlp_b2.reshape(1, M0 + M1 + M2)
    rpad = ((PAD, PAD), (0, 0))
    fpad = jnp.pad(features, rpad)
    cop = jnp.pad(coord, rpad)

    wcol = lambda shp: pl.BlockSpec(shp, lambda i: tuple(0 for _ in shp))
    out = pl.pallas_call(
        _fused_kernel,
        grid=(NB + 1,),
        in_specs=[
            wcol((NP2, D)), wcol((NP2, 3)), wcol((N, D)), wcol((N, 3)),
            wcol((M0, D + 1)), wcol((M1, D + 1)), wcol((M2, D + 1)),
            wcol((BINS, M0)), wcol((BINS, D)),
            wcol((BINS, BINS)), wcol((BINS, 1)),
            wcol((M0 + M1 + M2, BINS)), wcol((M0 + M1 + M2, 1)),
            wcol((3 * M1, M1)), wcol((3 * M1, 3)),
            wcol((5 * M2, M2)), wcol((5 * M2, 5)),
            wcol((D + 1, M0)), wcol((D + 1, M1)), wcol((D + 1, M2)),
            wcol((M0, BINS)), wcol((D, BINS)),
            wcol((BINS, BINS)), wcol((1, BINS)),
            wcol((BINS, M0 + M1 + M2)), wcol((1, M0 + M1 + M2)),
            wcol((M1, 3 * M1)), wcol((3, 3 * M1)),
            wcol((M2, 5 * M2)), wcol((5, 5 * M2)),
        ],
        out_specs=wcol((N, MOUT)),
        out_shape=jax.ShapeDtypeStruct((N, MOUT), f32),
    )(fpad, cop, features, coord,
      lin_w0.T, lin_w1.T, lin_w2.T, a_w.T, c_w.T,
      b_w.T, mlp_b1.reshape(BINS, 1), mlp_w2.T, mlp_b2.reshape(M0 + M1 + M2, 1),
      _R16.T, _S3M.T, _R8.T, _S5M.T,
      lin_w0, lin_w1, lin_w2, a_w, c_w,
      b_w, b1, mlp_w2, b2,
      _R16, _S3M, _R8, _S5M)
    return out
